# baseline (speedup 1.0000x reference)
import functools

import jax
import jax.numpy as jnp
from jax import lax
from jax.experimental import pallas as pl
from jax.experimental.pallas import tpu as pltpu
from jax.experimental.pallas import tpu_sc as plsc

NC, NS, L = 2, 16, 16
NW = NC * NS
B, N, K = 128, 100000, 32
RPW = B // NW
N2 = N // 2
CHUNK_V = 25
CHUNK_W = CHUNK_V * L
NCHUNK = N2 // CHUNK_W
NEG_INF = float("-inf")
CAPL = 392
FLUSH_AT = CAPL - 2 * CHUNK_V
HI_MASK = -65536


def _merge(T1, T2, v):
    vs = jnp.sort(v)
    rvs = lax.rev(vs, (0,))
    p = jnp.sort(jnp.maximum(T2, rvs))
    rp = lax.rev(p, (0,))
    T1n = jnp.sort(jnp.maximum(T1, rp))
    T2n = jnp.sort(jnp.minimum(T1, rp))
    return T1n, T2n


def _halves(w):
    lo = plsc.bitcast(lax.shift_left(w, 16), jnp.float32)
    hi = plsc.bitcast(jnp.bitwise_and(w, HI_MASK), jnp.float32)
    return lo, hi


def _row_topk(row_v, cm_s, cand_v):
    ninf = jnp.full((L,), NEG_INF, jnp.float32)

    @plsc.parallel_loop(0, NCHUNK, carry=(ninf, ninf))
    def passA(c, carry):
        m1, m2 = carry
        base = c * CHUNK_W
        lo, hi = _halves(row_v[pl.ds(base, L)])
        for k in range(1, CHUNK_V):
            l2, h2 = _halves(row_v[pl.ds(base + k * L, L)])
            lo = jnp.maximum(lo, l2)
            hi = jnp.maximum(hi, h2)
        am = jnp.maximum(lo, hi)
        cm_s[c] = jnp.max(am)
        m2n = jnp.maximum(m2, jnp.minimum(m1, am))
        m1n = jnp.maximum(m1, am)
        return (m1n, m2n)

    _, m2 = passA
    t0 = jnp.min(m2)
    t0q = jnp.full((L,), t0)
    iota = lax.iota(jnp.int32, L)
    zeros = jnp.zeros((L,), jnp.int32)

    def flush(T1, T2, olane):
        nmax = jnp.max(olane)

        def body(d, carry):
            T1, T2 = carry
            v = cand_v[pl.ds(d * L, L)]
            cand_v[pl.ds(d * L, L)] = ninf
            return _merge(T1, T2, v)

        T1, T2 = lax.fori_loop(0, nmax, body, (T1, T2))
        return T1, T2

    def phaseB(c, carry):
        def collect(carry):
            T1, T2, olane, wc = carry

            def maybe_flush(args):
                T1, T2, olane = args
                T1, T2 = flush(T1, T2, olane)
                return T1, T2, zeros

            T1, T2, olane = lax.cond(wc >= FLUSH_AT, maybe_flush,
                                     lambda a: a, (T1, T2, olane))
            wc = jnp.where(wc >= FLUSH_AT, 2 * CHUNK_V, wc + 2 * CHUNK_V)
            base = c * CHUNK_W
            for k in range(CHUNK_V):
                lo, hi = _halves(row_v[pl.ds(base + k * L, L)])
                for h in (lo, hi):
                    mask = h >= t0q
                    idx = lax.shift_left(olane, 4) + iota
                    plsc.store_scatter(cand_v, [idx], h, mask=mask)
                    olane = olane + jnp.where(mask, 1, 0)
            return T1, T2, olane, wc

        return lax.cond(cm_s[c] >= t0, collect, lambda q: q, carry)

    T1, T2, olane, _ = lax.fori_loop(
        0, NCHUNK, phaseB, (ninf, ninf, zeros, jnp.int32(0)))
    return flush(T1, T2, olane)


def _store_scalar(stage_v, idx, val_splat):
    mask = lax.iota(jnp.int32, L) == 0
    idxv = jnp.full((L,), idx, jnp.int32)
    plsc.store_scatter(stage_v, [idxv], val_splat, mask=mask)


def _sc_kernel(preds_hbm, labels_hbm, out_hbm, rowa_v, rowb_v, lab_v,
               stage_v, cm_s, cand_v, sems):
    wid = lax.axis_index("s") * NC + lax.axis_index("c")
    pltpu.sync_copy(labels_hbm, lab_v)
    ninf = jnp.full((L,), NEG_INF, jnp.float32)
    iota = lax.iota(jnp.int32, L)

    @plsc.parallel_loop(0, CAPL)
    def _(d):
        cand_v[pl.ds(d * L, L)] = ninf

    bufs = [rowa_v, rowb_v]
    r0 = wid * RPW
    descs = {0: pltpu.async_copy(preds_hbm.at[r0], bufs[0], sems.at[0])}
    for j in range(RPW):
        row_v = bufs[j % 2]
        descs[j].wait()
        if j + 1 < RPW:
            descs[j + 1] = pltpu.async_copy(
                preds_hbm.at[r0 + j + 1], bufs[(j + 1) % 2],
                sems.at[(j + 1) % 2])
        T1, T2 = _row_topk(row_v, cm_s, cand_v)
        m = jnp.max(T1)
        msplat = jnp.full((L,), m)
        s = jnp.sum(jnp.exp(T1 - msplat)) + jnp.sum(jnp.exp(T2 - msplat))
        lab_splat = plsc.load_gather(
            lab_v, [jnp.full((L,), r0 + j, jnp.int32)])
        p = jnp.max(lab_splat)
        wcol = p // 2
        vbase = (wcol // L) * L
        wv = row_v[pl.ds(vbase, L)]
        wsel = jnp.sum(jnp.where(iota == wcol - vbase, wv, 0))
        bits = jnp.where(p % 2 == 1, wsel & HI_MASK,
                         lax.shift_left(wsel, 16))
        neg_splat = plsc.bitcast(jnp.full((L,), bits), jnp.float32)
        _store_scalar(stage_v, j, msplat)
        _store_scalar(stage_v, RPW + j, jnp.full((L,), s))
        _store_scalar(stage_v, 2 * RPW + j, neg_splat)
    pltpu.sync_copy(stage_v, out_hbm.at[wid])


@functools.partial(jax.jit, static_argnames=())
def _sc_stage(preds_bf, labels32):
    mesh = plsc.VectorSubcoreMesh(core_axis_name="c", subcore_axis_name="s",
                                  num_cores=NC, num_subcores=NS)
    f = pl.kernel(
        _sc_kernel,
        out_type=jax.ShapeDtypeStruct((NW, 3 * RPW), jnp.float32),
        mesh=mesh,
        scratch_types=[
            pltpu.VMEM((N2,), jnp.int32),
            pltpu.VMEM((N2,), jnp.int32),
            pltpu.VMEM((B,), jnp.int32),
            pltpu.VMEM((3 * RPW,), jnp.float32),
            pltpu.SMEM((NCHUNK,), jnp.float32),
            pltpu.VMEM((CAPL * L,), jnp.float32),
            pltpu.SemaphoreType.DMA((2,)),
        ],
        compiler_params=pltpu.CompilerParams(needs_layout_passes=False),
    )
    return f(preds_bf, labels32)


def _tc_epilogue_kernel(x_ref, o_ref):
    x = x_ref[...]
    m = x[:, 0:RPW]
    s = x[:, RPW:2 * RPW]
    neg = x[:, 2 * RPW:3 * RPW]
    loss = m + jnp.log(s) - neg
    o_ref[0, 0] = jnp.mean(loss)


def _pack_bf16(preds):
    e = lax.bitcast_convert_type(
        preds[:, 0::2].astype(jnp.bfloat16), jnp.uint16).astype(jnp.uint32)
    o = lax.bitcast_convert_type(
        preds[:, 1::2].astype(jnp.bfloat16), jnp.uint16).astype(jnp.uint32)
    return lax.bitcast_convert_type(e | (o << 16), jnp.int32)


def kernel(preds, labels):
    labels32 = labels.astype(jnp.int32)
    stats = _sc_stage(_pack_bf16(preds), labels32)
    out = pl.pallas_call(
        _tc_epilogue_kernel,
        out_shape=jax.ShapeDtypeStruct((1, 1), jnp.float32),
        out_specs=pl.BlockSpec(memory_space=pltpu.SMEM),
    )(stats)
    return out.reshape(())


# speedup vs baseline: 2.2469x; 2.2469x over previous
import functools

import jax
import jax.numpy as jnp
from jax import lax
from jax.experimental import pallas as pl
from jax.experimental.pallas import tpu as pltpu
from jax.experimental.pallas import tpu_sc as plsc

NC, NS, L = 2, 16, 16
NW = NC * NS
B, N, K = 128, 100000, 32
RPW = B // NW
N2 = N // 2
CHUNK_V = 25
CHUNK_W = CHUNK_V * L
NCHUNK = N2 // CHUNK_W
NEG_INF = float("-inf")
CAPL = 392
FLUSH_AT = CAPL - 2 * CHUNK_V
HI_MASK = -65536


def _merge(T1, T2, v):
    vs = jnp.sort(v)
    rvs = lax.rev(vs, (0,))
    p = jnp.sort(jnp.maximum(T2, rvs))
    rp = lax.rev(p, (0,))
    T1n = jnp.sort(jnp.maximum(T1, rp))
    T2n = jnp.sort(jnp.minimum(T1, rp))
    return T1n, T2n


def _halves(w):
    lo = plsc.bitcast(lax.shift_left(w, 16), jnp.float32)
    hi = plsc.bitcast(jnp.bitwise_and(w, HI_MASK), jnp.float32)
    return lo, hi


def _row_topk(row_v, cm_s, cand_v):
    ninf = jnp.full((L,), NEG_INF, jnp.float32)

    @plsc.parallel_loop(0, NCHUNK, carry=(ninf, ninf))
    def passA(c, carry):
        m1, m2 = carry
        base = c * CHUNK_W
        lo, hi = _halves(row_v[pl.ds(base, L)])
        for k in range(1, CHUNK_V):
            l2, h2 = _halves(row_v[pl.ds(base + k * L, L)])
            lo = jnp.maximum(lo, l2)
            hi = jnp.maximum(hi, h2)
        am = jnp.maximum(lo, hi)
        cm_s[c] = jnp.max(am)
        m2n = jnp.maximum(m2, jnp.minimum(m1, am))
        m1n = jnp.maximum(m1, am)
        return (m1n, m2n)

    _, m2 = passA
    t0 = jnp.min(m2)
    t0q = jnp.full((L,), t0)
    iota = lax.iota(jnp.int32, L)
    zeros = jnp.zeros((L,), jnp.int32)

    def flush(T1, T2, olane):
        nmax = jnp.max(olane)

        def body(d, carry):
            T1, T2 = carry
            v = cand_v[pl.ds(d * L, L)]
            cand_v[pl.ds(d * L, L)] = ninf
            return _merge(T1, T2, v)

        T1, T2 = lax.fori_loop(0, nmax, body, (T1, T2))
        return T1, T2

    def phaseB(c, carry):
        def collect(carry):
            T1, T2, olane, wc = carry

            def maybe_flush(args):
                T1, T2, olane = args
                T1, T2 = flush(T1, T2, olane)
                return T1, T2, zeros

            T1, T2, olane = lax.cond(wc >= FLUSH_AT, maybe_flush,
                                     lambda a: a, (T1, T2, olane))
            wc = jnp.where(wc >= FLUSH_AT, 2 * CHUNK_V, wc + 2 * CHUNK_V)
            base = c * CHUNK_W
            for k in range(CHUNK_V):
                lo, hi = _halves(row_v[pl.ds(base + k * L, L)])
                for h in (lo, hi):
                    mask = h >= t0q
                    idx = lax.shift_left(olane, 4) + iota
                    plsc.store_scatter(cand_v, [idx], h, mask=mask)
                    olane = olane + jnp.where(mask, 1, 0)
            return T1, T2, olane, wc

        return lax.cond(cm_s[c] >= t0, collect, lambda q: q, carry)

    T1, T2, olane, _ = lax.fori_loop(
        0, NCHUNK, phaseB, (ninf, ninf, zeros, jnp.int32(0)))
    return flush(T1, T2, olane)


def _store_scalar(stage_v, idx, val_splat):
    mask = lax.iota(jnp.int32, L) == 0
    idxv = jnp.full((L,), idx, jnp.int32)
    plsc.store_scatter(stage_v, [idxv], val_splat, mask=mask)


def _sc_kernel(preds_hbm, labels_hbm, out_hbm, rowa_v, rowb_v, lab_v,
               stage_v, cm_s, cand_v, sems):
    wid = lax.axis_index("s") * NC + lax.axis_index("c")
    pltpu.sync_copy(labels_hbm, lab_v)
    ninf = jnp.full((L,), NEG_INF, jnp.float32)
    iota = lax.iota(jnp.int32, L)

    @plsc.parallel_loop(0, CAPL)
    def _(d):
        cand_v[pl.ds(d * L, L)] = ninf

    bufs = [rowa_v, rowb_v]
    r0 = wid * RPW
    descs = {0: pltpu.async_copy(preds_hbm.at[r0], bufs[0], sems.at[0])}
    for j in range(RPW):
        row_v = bufs[j % 2]
        descs[j].wait()
        if j + 1 < RPW:
            descs[j + 1] = pltpu.async_copy(
                preds_hbm.at[r0 + j + 1], bufs[(j + 1) % 2],
                sems.at[(j + 1) % 2])
        T1, T2 = _row_topk(row_v, cm_s, cand_v)
        m = jnp.max(T1)
        msplat = jnp.full((L,), m)
        s = jnp.sum(jnp.exp(T1 - msplat)) + jnp.sum(jnp.exp(T2 - msplat))
        lab_splat = plsc.load_gather(
            lab_v, [jnp.full((L,), r0 + j, jnp.int32)])
        p = jnp.max(lab_splat)
        wcol = jnp.where(p >= N2, p - N2, p)
        vbase = (wcol // L) * L
        wv = row_v[pl.ds(vbase, L)]
        wsel = jnp.sum(jnp.where(iota == wcol - vbase, wv, 0))
        bits = jnp.where(p >= N2, wsel & HI_MASK,
                         lax.shift_left(wsel, 16))
        neg_splat = plsc.bitcast(jnp.full((L,), bits), jnp.float32)
        _store_scalar(stage_v, j, msplat)
        _store_scalar(stage_v, RPW + j, jnp.full((L,), s))
        _store_scalar(stage_v, 2 * RPW + j, neg_splat)
    pltpu.sync_copy(stage_v, out_hbm.at[wid])


@functools.partial(jax.jit, static_argnames=())
def _sc_stage(preds_bf, labels32):
    mesh = plsc.VectorSubcoreMesh(core_axis_name="c", subcore_axis_name="s",
                                  num_cores=NC, num_subcores=NS)
    f = pl.kernel(
        _sc_kernel,
        out_type=jax.ShapeDtypeStruct((NW, 3 * RPW), jnp.float32),
        mesh=mesh,
        scratch_types=[
            pltpu.VMEM((N2,), jnp.int32),
            pltpu.VMEM((N2,), jnp.int32),
            pltpu.VMEM((B,), jnp.int32),
            pltpu.VMEM((3 * RPW,), jnp.float32),
            pltpu.SMEM((NCHUNK,), jnp.float32),
            pltpu.VMEM((CAPL * L,), jnp.float32),
            pltpu.SemaphoreType.DMA((2,)),
        ],
        compiler_params=pltpu.CompilerParams(needs_layout_passes=False),
    )
    return f(preds_bf, labels32)


def _tc_epilogue_kernel(x_ref, o_ref):
    x = x_ref[...]
    m = x[:, 0:RPW]
    s = x[:, RPW:2 * RPW]
    neg = x[:, 2 * RPW:3 * RPW]
    loss = m + jnp.log(s) - neg
    o_ref[0, 0] = jnp.mean(loss)


def _pack_bf16(preds):
    a = lax.bitcast_convert_type(
        preds[:, :N2].astype(jnp.bfloat16), jnp.uint16).astype(jnp.uint32)
    b = lax.bitcast_convert_type(
        preds[:, N2:].astype(jnp.bfloat16), jnp.uint16).astype(jnp.uint32)
    return lax.bitcast_convert_type(a | (b << 16), jnp.int32)


def kernel(preds, labels):
    labels32 = labels.astype(jnp.int32)
    stats = _sc_stage(_pack_bf16(preds), labels32)
    out = pl.pallas_call(
        _tc_epilogue_kernel,
        out_shape=jax.ShapeDtypeStruct((1, 1), jnp.float32),
        out_specs=pl.BlockSpec(memory_space=pltpu.SMEM),
    )(stats)
    return out.reshape(())


# speedup vs baseline: 2.4466x; 1.0889x over previous
import functools

import jax
import jax.numpy as jnp
from jax import lax
from jax.experimental import pallas as pl
from jax.experimental.pallas import tpu as pltpu
from jax.experimental.pallas import tpu_sc as plsc

NC, NS, L = 2, 16, 16
NW = NC * NS
B, N, K = 128, 100000, 32
RPW = B // NW
N2 = N // 2
CHUNK_V = 25
CHUNK_W = CHUNK_V * L
NCHUNK = N2 // CHUNK_W
NEG_INF = float("-inf")
CAPL = 640
HI_MASK = -65536


def _merge(T1, T2, v):
    vs = jnp.sort(v)
    rvs = lax.rev(vs, (0,))
    p = jnp.sort(jnp.maximum(T2, rvs))
    rp = lax.rev(p, (0,))
    T1n = jnp.sort(jnp.maximum(T1, rp))
    T2n = jnp.sort(jnp.minimum(T1, rp))
    return T1n, T2n


def _halves(w):
    lo = plsc.bitcast(lax.shift_left(w, 16), jnp.float32)
    hi = plsc.bitcast(jnp.bitwise_and(w, HI_MASK), jnp.float32)
    return lo, hi


def _row_topk(row_v, cm_s, candlo_v, candhi_v):
    ninf = jnp.full((L,), NEG_INF, jnp.float32)

    @plsc.parallel_loop(0, NCHUNK, carry=(ninf, ninf))
    def passA(c, carry):
        m1, m2 = carry
        base = c * CHUNK_W
        lo, hi = _halves(row_v[pl.ds(base, L)])
        for k in range(1, CHUNK_V):
            l2, h2 = _halves(row_v[pl.ds(base + k * L, L)])
            lo = jnp.maximum(lo, l2)
            hi = jnp.maximum(hi, h2)
        am = jnp.maximum(lo, hi)
        cm_s[c] = jnp.max(am)
        m2n = jnp.maximum(m2, jnp.minimum(m1, am))
        m1n = jnp.maximum(m1, am)
        return (m1n, m2n)

    _, m2 = passA
    t0 = jnp.min(m2)
    t0q = jnp.full((L,), t0)
    iota = lax.iota(jnp.int32, L)
    zeros = jnp.zeros((L,), jnp.int32)
    cap = jnp.full((L,), CAPL - 1, jnp.int32)

    def phaseB(c, olanes):
        def collect(olanes):
            ol, oh = olanes
            base = c * CHUNK_W
            for k in range(CHUNK_V):
                lo, hi = _halves(row_v[pl.ds(base + k * L, L)])
                mlo = lo >= t0q
                mhi = hi >= t0q
                ilo = lax.shift_left(jnp.minimum(ol, cap), 4) + iota
                ihi = lax.shift_left(jnp.minimum(oh, cap), 4) + iota
                plsc.store_scatter(candlo_v, [ilo], lo, mask=mlo)
                plsc.store_scatter(candhi_v, [ihi], hi, mask=mhi)
                ol = ol + jnp.where(mlo, 1, 0)
                oh = oh + jnp.where(mhi, 1, 0)
            return (ol, oh)

        return lax.cond(cm_s[c] >= t0, collect, lambda q: q, olanes)

    ol, oh = lax.fori_loop(0, NCHUNK, phaseB, (zeros, zeros))
    nlo = jnp.max(ol)
    nhi = jnp.max(oh)

    def merge_buf(buf_v, n, carry):
        def body(d, carry):
            T1, T2 = carry
            v = buf_v[pl.ds(d * L, L)]
            buf_v[pl.ds(d * L, L)] = ninf
            return _merge(T1, T2, v)

        return lax.fori_loop(0, n, body, carry)

    def fast(_):
        c1 = merge_buf(candlo_v, nlo, (ninf, ninf))
        return merge_buf(candhi_v, nhi, c1)

    def brute(_):
        def body(k, carry):
            T1, T2 = carry
            lo, hi = _halves(row_v[pl.ds(k * L, L)])
            return _merge(*_merge(T1, T2, lo), hi)

        c1 = lax.fori_loop(0, N2 // L, body, (ninf, ninf))
        c1 = merge_buf(candlo_v, jnp.minimum(nlo, CAPL), c1)
        return merge_buf(candhi_v, jnp.minimum(nhi, CAPL), c1)

    return lax.cond(jnp.maximum(nlo, nhi) > CAPL, brute, fast, 0)


def _store_scalar(stage_v, idx, val_splat):
    mask = lax.iota(jnp.int32, L) == 0
    idxv = jnp.full((L,), idx, jnp.int32)
    plsc.store_scatter(stage_v, [idxv], val_splat, mask=mask)


def _sc_kernel(preds_hbm, labels_hbm, out_hbm, rowa_v, rowb_v, lab_v,
               stage_v, cm_s, candlo_v, candhi_v, sems):
    wid = lax.axis_index("s") * NC + lax.axis_index("c")
    pltpu.sync_copy(labels_hbm, lab_v)
    ninf = jnp.full((L,), NEG_INF, jnp.float32)
    iota = lax.iota(jnp.int32, L)

    @plsc.parallel_loop(0, CAPL)
    def _(d):
        candlo_v[pl.ds(d * L, L)] = ninf
        candhi_v[pl.ds(d * L, L)] = ninf

    bufs = [rowa_v, rowb_v]
    r0 = wid * RPW
    descs = {0: pltpu.async_copy(preds_hbm.at[r0], bufs[0], sems.at[0])}
    for j in range(RPW):
        row_v = bufs[j % 2]
        descs[j].wait()
        if j + 1 < RPW:
            descs[j + 1] = pltpu.async_copy(
                preds_hbm.at[r0 + j + 1], bufs[(j + 1) % 2],
                sems.at[(j + 1) % 2])
        T1, T2 = _row_topk(row_v, cm_s, candlo_v, candhi_v)
        m = jnp.max(T1)
        msplat = jnp.full((L,), m)
        s = jnp.sum(jnp.exp(T1 - msplat)) + jnp.sum(jnp.exp(T2 - msplat))
        lab_splat = plsc.load_gather(
            lab_v, [jnp.full((L,), r0 + j, jnp.int32)])
        p = jnp.max(lab_splat)
        wcol = jnp.where(p >= N2, p - N2, p)
        vbase = (wcol // L) * L
        wv = row_v[pl.ds(vbase, L)]
        wsel = jnp.sum(jnp.where(iota == wcol - vbase, wv, 0))
        bits = jnp.where(p >= N2, wsel & HI_MASK,
                         lax.shift_left(wsel, 16))
        neg_splat = plsc.bitcast(jnp.full((L,), bits), jnp.float32)
        _store_scalar(stage_v, j, msplat)
        _store_scalar(stage_v, RPW + j, jnp.full((L,), s))
        _store_scalar(stage_v, 2 * RPW + j, neg_splat)
    pltpu.sync_copy(stage_v, out_hbm.at[wid])


@functools.partial(jax.jit, static_argnames=())
def _sc_stage(preds_bf, labels32):
    mesh = plsc.VectorSubcoreMesh(core_axis_name="c", subcore_axis_name="s",
                                  num_cores=NC, num_subcores=NS)
    f = pl.kernel(
        _sc_kernel,
        out_type=jax.ShapeDtypeStruct((NW, 3 * RPW), jnp.float32),
        mesh=mesh,
        scratch_types=[
            pltpu.VMEM((N2,), jnp.int32),
            pltpu.VMEM((N2,), jnp.int32),
            pltpu.VMEM((B,), jnp.int32),
            pltpu.VMEM((3 * RPW,), jnp.float32),
            pltpu.SMEM((NCHUNK,), jnp.float32),
            pltpu.VMEM((CAPL * L,), jnp.float32),
            pltpu.VMEM((CAPL * L,), jnp.float32),
            pltpu.SemaphoreType.DMA((2,)),
        ],
        compiler_params=pltpu.CompilerParams(needs_layout_passes=False),
    )
    return f(preds_bf, labels32)


def _tc_epilogue_kernel(x_ref, o_ref):
    x = x_ref[...]
    m = x[:, 0:RPW]
    s = x[:, RPW:2 * RPW]
    neg = x[:, 2 * RPW:3 * RPW]
    loss = m + jnp.log(s) - neg
    o_ref[0, 0] = jnp.mean(loss)


def _pack_bf16(preds):
    a = lax.bitcast_convert_type(
        preds[:, :N2].astype(jnp.bfloat16), jnp.uint16).astype(jnp.uint32)
    b = lax.bitcast_convert_type(
        preds[:, N2:].astype(jnp.bfloat16), jnp.uint16).astype(jnp.uint32)
    return lax.bitcast_convert_type(a | (b << 16), jnp.int32)


def kernel(preds, labels):
    labels32 = labels.astype(jnp.int32)
    stats = _sc_stage(_pack_bf16(preds), labels32)
    out = pl.pallas_call(
        _tc_epilogue_kernel,
        out_shape=jax.ShapeDtypeStruct((1, 1), jnp.float32),
        out_specs=pl.BlockSpec(memory_space=pltpu.SMEM),
    )(stats)
    return out.reshape(())


# speedup vs baseline: 2.4648x; 1.0074x over previous
import functools

import jax
import jax.numpy as jnp
from jax import lax
from jax.experimental import pallas as pl
from jax.experimental.pallas import tpu as pltpu
from jax.experimental.pallas import tpu_sc as plsc

NC, NS, L = 2, 16, 16
NW = NC * NS
B, N, K = 128, 100000, 32
RPW = B // NW
N2 = N // 2
CHUNK_V = 25
CHUNK_W = CHUNK_V * L
NCHUNK = N2 // CHUNK_W
NEG_INF = float("-inf")
CAPL = 640
HI_MASK = -65536
SPL = 49920
TAIL = (N - 2 * SPL) // 2


def _merge(T1, T2, v):
    vs = jnp.sort(v)
    rvs = lax.rev(vs, (0,))
    p = jnp.sort(jnp.maximum(T2, rvs))
    rp = lax.rev(p, (0,))
    T1n = jnp.sort(jnp.maximum(T1, rp))
    T2n = jnp.sort(jnp.minimum(T1, rp))
    return T1n, T2n


def _halves(w):
    lo = plsc.bitcast(lax.shift_left(w, 16), jnp.float32)
    hi = plsc.bitcast(jnp.bitwise_and(w, HI_MASK), jnp.float32)
    return lo, hi


def _row_topk(row_v, cm_s, candlo_v, candhi_v):
    ninf = jnp.full((L,), NEG_INF, jnp.float32)

    @plsc.parallel_loop(0, NCHUNK, carry=(ninf, ninf))
    def passA(c, carry):
        m1, m2 = carry
        base = c * CHUNK_W
        lo, hi = _halves(row_v[pl.ds(base, L)])
        for k in range(1, CHUNK_V):
            l2, h2 = _halves(row_v[pl.ds(base + k * L, L)])
            lo = jnp.maximum(lo, l2)
            hi = jnp.maximum(hi, h2)
        am = jnp.maximum(lo, hi)
        cm_s[c] = jnp.max(am)
        m2n = jnp.maximum(m2, jnp.minimum(m1, am))
        m1n = jnp.maximum(m1, am)
        return (m1n, m2n)

    _, m2 = passA
    t0 = jnp.min(m2)
    t0q = jnp.full((L,), t0)
    iota = lax.iota(jnp.int32, L)
    zeros = jnp.zeros((L,), jnp.int32)
    cap = jnp.full((L,), CAPL - 1, jnp.int32)

    def phaseB(c, olanes):
        def collect(olanes):
            ol, oh = olanes
            base = c * CHUNK_W
            for k in range(CHUNK_V):
                lo, hi = _halves(row_v[pl.ds(base + k * L, L)])
                mlo = lo >= t0q
                mhi = hi >= t0q
                ilo = lax.shift_left(jnp.minimum(ol, cap), 4) + iota
                ihi = lax.shift_left(jnp.minimum(oh, cap), 4) + iota
                plsc.store_scatter(candlo_v, [ilo], lo, mask=mlo)
                plsc.store_scatter(candhi_v, [ihi], hi, mask=mhi)
                ol = ol + jnp.where(mlo, 1, 0)
                oh = oh + jnp.where(mhi, 1, 0)
            return (ol, oh)

        return lax.cond(cm_s[c] >= t0, collect, lambda q: q, olanes)

    ol, oh = lax.fori_loop(0, NCHUNK, phaseB, (zeros, zeros))
    nlo = jnp.max(ol)
    nhi = jnp.max(oh)

    def merge_buf(buf_v, n, carry):
        def body(d, carry):
            T1, T2 = carry
            v = buf_v[pl.ds(d * L, L)]
            buf_v[pl.ds(d * L, L)] = ninf
            return _merge(T1, T2, v)

        return lax.fori_loop(0, n, body, carry)

    def fast(_):
        c1 = merge_buf(candlo_v, nlo, (ninf, ninf))
        return merge_buf(candhi_v, nhi, c1)

    def brute(_):
        def body(k, carry):
            T1, T2 = carry
            lo, hi = _halves(row_v[pl.ds(k * L, L)])
            return _merge(*_merge(T1, T2, lo), hi)

        c1 = lax.fori_loop(0, N2 // L, body, (ninf, ninf))
        c1 = merge_buf(candlo_v, jnp.minimum(nlo, CAPL), c1)
        return merge_buf(candhi_v, jnp.minimum(nhi, CAPL), c1)

    return lax.cond(jnp.maximum(nlo, nhi) > CAPL, brute, fast, 0)


def _store_scalar(stage_v, idx, val_splat):
    mask = lax.iota(jnp.int32, L) == 0
    idxv = jnp.full((L,), idx, jnp.int32)
    plsc.store_scatter(stage_v, [idxv], val_splat, mask=mask)


def _sc_kernel(preds_hbm, labels_hbm, out_hbm, rowa_v, rowb_v, lab_v,
               stage_v, cm_s, candlo_v, candhi_v, sems):
    wid = lax.axis_index("s") * NC + lax.axis_index("c")
    pltpu.sync_copy(labels_hbm, lab_v)
    ninf = jnp.full((L,), NEG_INF, jnp.float32)
    iota = lax.iota(jnp.int32, L)

    @plsc.parallel_loop(0, CAPL)
    def _(d):
        candlo_v[pl.ds(d * L, L)] = ninf
        candhi_v[pl.ds(d * L, L)] = ninf

    bufs = [rowa_v, rowb_v]
    r0 = wid * RPW
    descs = {0: pltpu.async_copy(preds_hbm.at[r0], bufs[0], sems.at[0])}
    for j in range(RPW):
        row_v = bufs[j % 2]
        descs[j].wait()
        if j + 1 < RPW:
            descs[j + 1] = pltpu.async_copy(
                preds_hbm.at[r0 + j + 1], bufs[(j + 1) % 2],
                sems.at[(j + 1) % 2])
        T1, T2 = _row_topk(row_v, cm_s, candlo_v, candhi_v)
        m = jnp.max(T1)
        msplat = jnp.full((L,), m)
        s = jnp.sum(jnp.exp(T1 - msplat)) + jnp.sum(jnp.exp(T2 - msplat))
        lab_splat = plsc.load_gather(
            lab_v, [jnp.full((L,), r0 + j, jnp.int32)])
        p = jnp.max(lab_splat)
        in_main = p < 2 * SPL
        in_lo = jnp.where(in_main, (p % (2 * SPL)) < SPL,
                          ((p - 2 * SPL) % (2 * TAIL)) < TAIL)
        wcol = jnp.where(
            in_main, jnp.where(p < SPL, p, p - SPL),
            SPL + jnp.where(p < 2 * SPL + TAIL, p - 2 * SPL,
                            p - 2 * SPL - TAIL))
        vbase = (wcol // L) * L
        wv = row_v[pl.ds(vbase, L)]
        wsel = jnp.sum(jnp.where(iota == wcol - vbase, wv, 0))
        bits = jnp.where(in_lo, lax.shift_left(wsel, 16), wsel & HI_MASK)
        neg_splat = plsc.bitcast(jnp.full((L,), bits), jnp.float32)
        _store_scalar(stage_v, j, msplat)
        _store_scalar(stage_v, RPW + j, jnp.full((L,), s))
        _store_scalar(stage_v, 2 * RPW + j, neg_splat)
    pltpu.sync_copy(stage_v, out_hbm.at[wid])


@functools.partial(jax.jit, static_argnames=())
def _sc_stage(preds_bf, labels32):
    mesh = plsc.VectorSubcoreMesh(core_axis_name="c", subcore_axis_name="s",
                                  num_cores=NC, num_subcores=NS)
    f = pl.kernel(
        _sc_kernel,
        out_type=jax.ShapeDtypeStruct((NW, 3 * RPW), jnp.float32),
        mesh=mesh,
        scratch_types=[
            pltpu.VMEM((N2,), jnp.int32),
            pltpu.VMEM((N2,), jnp.int32),
            pltpu.VMEM((B,), jnp.int32),
            pltpu.VMEM((3 * RPW,), jnp.float32),
            pltpu.SMEM((NCHUNK,), jnp.float32),
            pltpu.VMEM((CAPL * L,), jnp.float32),
            pltpu.VMEM((CAPL * L,), jnp.float32),
            pltpu.SemaphoreType.DMA((2,)),
        ],
        compiler_params=pltpu.CompilerParams(needs_layout_passes=False),
    )
    return f(preds_bf, labels32)


def _tc_epilogue_kernel(x_ref, o_ref):
    x = x_ref[...]
    m = x[:, 0:RPW]
    s = x[:, RPW:2 * RPW]
    neg = x[:, 2 * RPW:3 * RPW]
    loss = m + jnp.log(s) - neg
    o_ref[0, 0] = jnp.mean(loss)


def _pack2(a, b):
    pa = lax.shift_right_logical(
        lax.bitcast_convert_type(
            a.astype(jnp.bfloat16).astype(jnp.float32), jnp.uint32), jnp.uint32(16))
    pb = jnp.bitwise_and(
        lax.bitcast_convert_type(
            b.astype(jnp.bfloat16).astype(jnp.float32), jnp.uint32),
        jnp.uint32(0xFFFF0000)).astype(jnp.uint32)
    return lax.bitcast_convert_type(pa | pb, jnp.int32)


def _pack_bf16(preds):
    main = _pack2(preds[:, :SPL], preds[:, SPL:2 * SPL])
    tail = _pack2(preds[:, 2 * SPL:2 * SPL + TAIL],
                  preds[:, 2 * SPL + TAIL:])
    return jnp.concatenate([main, tail], axis=1)


def kernel(preds, labels):
    labels32 = labels.astype(jnp.int32)
    stats = _sc_stage(_pack_bf16(preds), labels32)
    out = pl.pallas_call(
        _tc_epilogue_kernel,
        out_shape=jax.ShapeDtypeStruct((1, 1), jnp.float32),
        out_specs=pl.BlockSpec(memory_space=pltpu.SMEM),
    )(stats)
    return out.reshape(())


# speedup vs baseline: 3.0789x; 1.2492x over previous
import functools

import jax
import jax.numpy as jnp
from jax import lax
from jax.experimental import pallas as pl
from jax.experimental.pallas import tpu as pltpu
from jax.experimental.pallas import tpu_sc as plsc

NC, NS, L = 2, 16, 16
NW = NC * NS
B, N, K = 128, 100000, 32
RPW = B // NW
N2 = N // 2
CHUNK_V = 25
CHUNK_W = CHUNK_V * L
NCHUNK = N2 // CHUNK_W
NEG_INF = float("-inf")
CAPL = 640
NCMV = 128
HI_MASK = -65536
SPL = 49920
TAIL = (N - 2 * SPL) // 2


def _merge(T1, T2, v):
    vs = jnp.sort(v)
    rvs = lax.rev(vs, (0,))
    p = jnp.sort(jnp.maximum(T2, rvs))
    rp = lax.rev(p, (0,))
    T1n = jnp.sort(jnp.maximum(T1, rp))
    T2n = jnp.sort(jnp.minimum(T1, rp))
    return T1n, T2n


def _halves(w):
    lo = plsc.bitcast(lax.shift_left(w, 16), jnp.float32)
    hi = plsc.bitcast(jnp.bitwise_and(w, HI_MASK), jnp.float32)
    return lo, hi


def _row_topk(row_v, cm_s, cmv_v, candlo_v, candhi_v):
    ninf = jnp.full((L,), NEG_INF, jnp.float32)

    @plsc.parallel_loop(0, NCHUNK, carry=(ninf, ninf))
    def passA(c, carry):
        m1, m2 = carry
        base = c * CHUNK_W
        lo, hi = _halves(row_v[pl.ds(base, L)])
        for k in range(1, CHUNK_V):
            l2, h2 = _halves(row_v[pl.ds(base + k * L, L)])
            lo = jnp.maximum(lo, l2)
            hi = jnp.maximum(hi, h2)
        am = jnp.maximum(lo, hi)
        cmax = jnp.max(am)
        cm_s[c] = cmax
        ci = jnp.full((L,), c, jnp.int32)
        plsc.store_scatter(cmv_v, [ci],
                           jnp.full((L,), cmax),
                           mask=lax.iota(jnp.int32, L) == (c % L))
        m2n = jnp.maximum(m2, jnp.minimum(m1, am))
        m1n = jnp.maximum(m1, am)
        return (m1n, m2n)

    _, m2 = passA
    ninf2 = jnp.full((L,), NEG_INF, jnp.float32)
    c1 = (ninf2, ninf2)
    for g in range(NCMV // L):
        c1 = _merge(*c1, cmv_v[pl.ds(g * L, L)])
    t0 = jnp.maximum(jnp.min(m2), jnp.min(c1[1]))
    t0q = jnp.full((L,), t0)
    iota = lax.iota(jnp.int32, L)
    zeros = jnp.zeros((L,), jnp.int32)
    cap = jnp.full((L,), CAPL - 1, jnp.int32)

    def phaseB(c, olanes):
        def collect(olanes):
            ol, oh = olanes
            base = c * CHUNK_W
            for k in range(CHUNK_V):
                lo, hi = _halves(row_v[pl.ds(base + k * L, L)])
                mlo = lo >= t0q
                mhi = hi >= t0q
                ilo = lax.shift_left(jnp.minimum(ol, cap), 4) + iota
                ihi = lax.shift_left(jnp.minimum(oh, cap), 4) + iota
                plsc.store_scatter(candlo_v, [ilo], lo, mask=mlo)
                plsc.store_scatter(candhi_v, [ihi], hi, mask=mhi)
                ol = ol + jnp.where(mlo, 1, 0)
                oh = oh + jnp.where(mhi, 1, 0)
            return (ol, oh)

        return lax.cond(cm_s[c] >= t0, collect, lambda q: q, olanes)

    ol, oh = lax.fori_loop(0, NCHUNK, phaseB, (zeros, zeros))
    nlo = jnp.max(ol)
    nhi = jnp.max(oh)

    def merge_buf(buf_v, n, carry):
        def body(d, carry):
            T1, T2 = carry
            v = buf_v[pl.ds(d * L, L)]
            buf_v[pl.ds(d * L, L)] = ninf
            return _merge(T1, T2, v)

        return lax.fori_loop(0, n, body, carry)

    def fast(_):
        c1 = merge_buf(candlo_v, nlo, (ninf, ninf))
        return merge_buf(candhi_v, nhi, c1)

    def brute(_):
        def body(k, carry):
            T1, T2 = carry
            lo, hi = _halves(row_v[pl.ds(k * L, L)])
            return _merge(*_merge(T1, T2, lo), hi)

        c1 = lax.fori_loop(0, N2 // L, body, (ninf, ninf))
        c1 = merge_buf(candlo_v, jnp.minimum(nlo, CAPL), c1)
        return merge_buf(candhi_v, jnp.minimum(nhi, CAPL), c1)

    return lax.cond(jnp.maximum(nlo, nhi) > CAPL, brute, fast, 0)


def _store_scalar(stage_v, idx, val_splat):
    mask = lax.iota(jnp.int32, L) == 0
    idxv = jnp.full((L,), idx, jnp.int32)
    plsc.store_scatter(stage_v, [idxv], val_splat, mask=mask)


def _sc_kernel(preds_hbm, labels_hbm, out_hbm, rowa_v, rowb_v, lab_v,
               stage_v, cm_s, cmv_v, candlo_v, candhi_v, sems):
    wid = lax.axis_index("s") * NC + lax.axis_index("c")
    pltpu.sync_copy(labels_hbm, lab_v)
    ninf = jnp.full((L,), NEG_INF, jnp.float32)
    iota = lax.iota(jnp.int32, L)

    @plsc.parallel_loop(0, CAPL)
    def _(d):
        candlo_v[pl.ds(d * L, L)] = ninf
        candhi_v[pl.ds(d * L, L)] = ninf

    cmv_v[pl.ds(NCMV - L, L)] = ninf

    bufs = [rowa_v, rowb_v]
    r0 = wid * RPW
    descs = {0: pltpu.async_copy(preds_hbm.at[r0], bufs[0], sems.at[0])}
    for j in range(RPW):
        row_v = bufs[j % 2]
        descs[j].wait()
        if j + 1 < RPW:
            descs[j + 1] = pltpu.async_copy(
                preds_hbm.at[r0 + j + 1], bufs[(j + 1) % 2],
                sems.at[(j + 1) % 2])
        T1, T2 = _row_topk(row_v, cm_s, cmv_v, candlo_v, candhi_v)
        m = jnp.max(T1)
        msplat = jnp.full((L,), m)
        s = jnp.sum(jnp.exp(T1 - msplat)) + jnp.sum(jnp.exp(T2 - msplat))
        lab_splat = plsc.load_gather(
            lab_v, [jnp.full((L,), r0 + j, jnp.int32)])
        p = jnp.max(lab_splat)
        in_main = p < 2 * SPL
        in_lo = jnp.where(in_main, (p % (2 * SPL)) < SPL,
                          ((p - 2 * SPL) % (2 * TAIL)) < TAIL)
        wcol = jnp.where(
            in_main, jnp.where(p < SPL, p, p - SPL),
            SPL + jnp.where(p < 2 * SPL + TAIL, p - 2 * SPL,
                            p - 2 * SPL - TAIL))
        vbase = (wcol // L) * L
        wv = row_v[pl.ds(vbase, L)]
        wsel = jnp.sum(jnp.where(iota == wcol - vbase, wv, 0))
        bits = jnp.where(in_lo, lax.shift_left(wsel, 16), wsel & HI_MASK)
        neg_splat = plsc.bitcast(jnp.full((L,), bits), jnp.float32)
        _store_scalar(stage_v, j, msplat)
        _store_scalar(stage_v, RPW + j, jnp.full((L,), s))
        _store_scalar(stage_v, 2 * RPW + j, neg_splat)
    pltpu.sync_copy(stage_v, out_hbm.at[wid])


@functools.partial(jax.jit, static_argnames=())
def _sc_stage(preds_bf, labels32):
    mesh = plsc.VectorSubcoreMesh(core_axis_name="c", subcore_axis_name="s",
                                  num_cores=NC, num_subcores=NS)
    f = pl.kernel(
        _sc_kernel,
        out_type=jax.ShapeDtypeStruct((NW, 3 * RPW), jnp.float32),
        mesh=mesh,
        scratch_types=[
            pltpu.VMEM((N2,), jnp.int32),
            pltpu.VMEM((N2,), jnp.int32),
            pltpu.VMEM((B,), jnp.int32),
            pltpu.VMEM((3 * RPW,), jnp.float32),
            pltpu.SMEM((NCHUNK,), jnp.float32),
            pltpu.VMEM((NCMV,), jnp.float32),
            pltpu.VMEM((CAPL * L,), jnp.float32),
            pltpu.VMEM((CAPL * L,), jnp.float32),
            pltpu.SemaphoreType.DMA((2,)),
        ],
        compiler_params=pltpu.CompilerParams(needs_layout_passes=False),
    )
    return f(preds_bf, labels32)


def _tc_epilogue_kernel(x_ref, o_ref):
    x = x_ref[...]
    m = x[:, 0:RPW]
    s = x[:, RPW:2 * RPW]
    neg = x[:, 2 * RPW:3 * RPW]
    loss = m + jnp.log(s) - neg
    o_ref[0, 0] = jnp.mean(loss)


def _pack2(a, b):
    pa = lax.shift_right_logical(
        lax.bitcast_convert_type(
            a.astype(jnp.bfloat16).astype(jnp.float32), jnp.uint32), jnp.uint32(16))
    pb = jnp.bitwise_and(
        lax.bitcast_convert_type(
            b.astype(jnp.bfloat16).astype(jnp.float32), jnp.uint32),
        jnp.uint32(0xFFFF0000)).astype(jnp.uint32)
    return lax.bitcast_convert_type(pa | pb, jnp.int32)


def _pack_bf16(preds):
    main = _pack2(preds[:, :SPL], preds[:, SPL:2 * SPL])
    tail = _pack2(preds[:, 2 * SPL:2 * SPL + TAIL],
                  preds[:, 2 * SPL + TAIL:])
    return jnp.concatenate([main, tail], axis=1)


def kernel(preds, labels):
    labels32 = labels.astype(jnp.int32)
    stats = _sc_stage(_pack_bf16(preds), labels32)
    out = pl.pallas_call(
        _tc_epilogue_kernel,
        out_shape=jax.ShapeDtypeStruct((1, 1), jnp.float32),
        out_specs=pl.BlockSpec(memory_space=pltpu.SMEM),
    )(stats)
    return out.reshape(())


# speedup vs baseline: 3.1132x; 1.0112x over previous
import functools

import jax
import jax.numpy as jnp
from jax import lax
from jax.experimental import pallas as pl
from jax.experimental.pallas import tpu as pltpu
from jax.experimental.pallas import tpu_sc as plsc

NC, NS, L = 2, 16, 16
NW = NC * NS
B, N, K = 128, 100000, 32
RPW = B // NW
N2 = N // 2
CHUNK_V = 25
CHUNK_W = CHUNK_V * L
NCHUNK = N2 // CHUNK_W
NEG_INF = float("-inf")
CAPL = 640
NCMV = 128
HI_MASK = -65536
SPL = 49920
TAIL = (N - 2 * SPL) // 2


def _merge(T1, T2, v):
    vs = jnp.sort(v)
    rvs = lax.rev(vs, (0,))
    p = jnp.sort(jnp.maximum(T2, rvs))
    rp = lax.rev(p, (0,))
    T1n = jnp.sort(jnp.maximum(T1, rp))
    T2n = jnp.sort(jnp.minimum(T1, rp))
    return T1n, T2n


def _halves(w):
    lo = plsc.bitcast(lax.shift_left(w, 16), jnp.float32)
    hi = plsc.bitcast(jnp.bitwise_and(w, HI_MASK), jnp.float32)
    return lo, hi


def _row_topk(row_v, cm_s, cmv_v, candlo_v, candhi_v):
    ninf = jnp.full((L,), NEG_INF, jnp.float32)

    @plsc.parallel_loop(0, NCHUNK, carry=(ninf, ninf))
    def passA(c, carry):
        m1, m2 = carry
        base = c * CHUNK_W
        lo, hi = _halves(row_v[pl.ds(base, L)])
        for k in range(1, CHUNK_V):
            l2, h2 = _halves(row_v[pl.ds(base + k * L, L)])
            lo = jnp.maximum(lo, l2)
            hi = jnp.maximum(hi, h2)
        am = jnp.maximum(lo, hi)
        cmax = jnp.max(am)
        cm_s[c] = cmax
        ci = jnp.full((L,), c, jnp.int32)
        plsc.store_scatter(cmv_v, [ci],
                           jnp.full((L,), cmax),
                           mask=lax.iota(jnp.int32, L) == (c % L))
        m2n = jnp.maximum(m2, jnp.minimum(m1, am))
        m1n = jnp.maximum(m1, am)
        return (m1n, m2n)

    _, m2 = passA
    ninf2 = jnp.full((L,), NEG_INF, jnp.float32)
    c1 = (ninf2, ninf2)
    for g in range(NCMV // L):
        c1 = _merge(*c1, cmv_v[pl.ds(g * L, L)])
    t0 = jnp.maximum(jnp.min(m2), jnp.min(c1[1]))
    t0q = jnp.full((L,), t0)
    iota = lax.iota(jnp.int32, L)
    zeros = jnp.zeros((L,), jnp.int32)
    cap = jnp.full((L,), CAPL - 1, jnp.int32)

    def phaseB(c, olanes):
        def collect(olanes):
            ol, oh = olanes
            base = c * CHUNK_W
            for k in range(CHUNK_V):
                lo, hi = _halves(row_v[pl.ds(base + k * L, L)])
                mlo = lo >= t0q
                mhi = hi >= t0q
                ilo = lax.shift_left(jnp.minimum(ol, cap), 4) + iota
                ihi = lax.shift_left(jnp.minimum(oh, cap), 4) + iota
                plsc.store_scatter(candlo_v, [ilo], lo, mask=mlo)
                plsc.store_scatter(candhi_v, [ihi], hi, mask=mhi)
                ol = ol + jnp.where(mlo, 1, 0)
                oh = oh + jnp.where(mhi, 1, 0)
            return (ol, oh)

        return lax.cond(cm_s[c] >= t0, collect, lambda q: q, olanes)

    ol, oh = lax.fori_loop(0, NCHUNK, phaseB, (zeros, zeros))
    nlo = jnp.max(ol)
    nhi = jnp.max(oh)

    def merge_buf(buf_v, n, carry):
        def body(d, carry):
            T1, T2 = carry
            v = buf_v[pl.ds(d * L, L)]
            buf_v[pl.ds(d * L, L)] = ninf
            return _merge(T1, T2, v)

        return lax.fori_loop(0, n, body, carry)

    def fast(_):
        c1 = merge_buf(candlo_v, nlo, (ninf, ninf))
        return merge_buf(candhi_v, nhi, c1)

    def brute(_):
        def body(k, carry):
            T1, T2 = carry
            lo, hi = _halves(row_v[pl.ds(k * L, L)])
            return _merge(*_merge(T1, T2, lo), hi)

        c1 = lax.fori_loop(0, N2 // L, body, (ninf, ninf))
        c1 = merge_buf(candlo_v, jnp.minimum(nlo, CAPL), c1)
        return merge_buf(candhi_v, jnp.minimum(nhi, CAPL), c1)

    return lax.cond(jnp.maximum(nlo, nhi) > CAPL, brute, fast, 0)


def _store_scalar(stage_v, idx, val_splat):
    mask = lax.iota(jnp.int32, L) == 0
    idxv = jnp.full((L,), idx, jnp.int32)
    plsc.store_scatter(stage_v, [idxv], val_splat, mask=mask)


def _sc_kernel(preds_hbm, labels_hbm, out_hbm, rowa_v, rowb_v, lab_v,
               stage_v, cm_s, cmv_v, candlo_v, candhi_v, sems):
    wid = lax.axis_index("s") * NC + lax.axis_index("c")
    ninf = jnp.full((L,), NEG_INF, jnp.float32)
    iota = lax.iota(jnp.int32, L)
    bufs = [rowa_v, rowb_v]
    r0 = wid * RPW
    descs = {0: pltpu.async_copy(preds_hbm.at[r0], bufs[0], sems.at[0]),
             1: pltpu.async_copy(preds_hbm.at[r0 + 1], bufs[1], sems.at[1])}
    pltpu.sync_copy(labels_hbm, lab_v)

    @plsc.parallel_loop(0, CAPL)
    def _(d):
        candlo_v[pl.ds(d * L, L)] = ninf
        candhi_v[pl.ds(d * L, L)] = ninf

    cmv_v[pl.ds(NCMV - L, L)] = ninf

    for j in range(RPW):
        row_v = bufs[j % 2]
        descs[j].wait()
        T1, T2 = _row_topk(row_v, cm_s, cmv_v, candlo_v, candhi_v)
        m = jnp.max(T1)
        msplat = jnp.full((L,), m)
        s = jnp.sum(jnp.exp(T1 - msplat)) + jnp.sum(jnp.exp(T2 - msplat))
        lab_splat = plsc.load_gather(
            lab_v, [jnp.full((L,), r0 + j, jnp.int32)])
        p = jnp.max(lab_splat)
        in_main = p < 2 * SPL
        in_lo = jnp.where(in_main, (p % (2 * SPL)) < SPL,
                          ((p - 2 * SPL) % (2 * TAIL)) < TAIL)
        wcol = jnp.where(
            in_main, jnp.where(p < SPL, p, p - SPL),
            SPL + jnp.where(p < 2 * SPL + TAIL, p - 2 * SPL,
                            p - 2 * SPL - TAIL))
        vbase = (wcol // L) * L
        wv = row_v[pl.ds(vbase, L)]
        wsel = jnp.sum(jnp.where(iota == wcol - vbase, wv, 0))
        bits = jnp.where(in_lo, lax.shift_left(wsel, 16), wsel & HI_MASK)
        neg_splat = plsc.bitcast(jnp.full((L,), bits), jnp.float32)
        _store_scalar(stage_v, j, msplat)
        _store_scalar(stage_v, RPW + j, jnp.full((L,), s))
        _store_scalar(stage_v, 2 * RPW + j, neg_splat)
        if j + 2 < RPW:
            descs[j + 2] = pltpu.async_copy(
                preds_hbm.at[r0 + j + 2], bufs[j % 2], sems.at[j % 2])
    pltpu.sync_copy(stage_v, out_hbm.at[wid])


@functools.partial(jax.jit, static_argnames=())
def _sc_stage(preds_bf, labels32):
    mesh = plsc.VectorSubcoreMesh(core_axis_name="c", subcore_axis_name="s",
                                  num_cores=NC, num_subcores=NS)
    f = pl.kernel(
        _sc_kernel,
        out_type=jax.ShapeDtypeStruct((NW, 3 * RPW), jnp.float32),
        mesh=mesh,
        scratch_types=[
            pltpu.VMEM((N2,), jnp.int32),
            pltpu.VMEM((N2,), jnp.int32),
            pltpu.VMEM((B,), jnp.int32),
            pltpu.VMEM((3 * RPW,), jnp.float32),
            pltpu.SMEM((NCHUNK,), jnp.float32),
            pltpu.VMEM((NCMV,), jnp.float32),
            pltpu.VMEM((CAPL * L,), jnp.float32),
            pltpu.VMEM((CAPL * L,), jnp.float32),
            pltpu.SemaphoreType.DMA((2,)),
        ],
        compiler_params=pltpu.CompilerParams(needs_layout_passes=False),
    )
    return f(preds_bf, labels32)


def _tc_epilogue_kernel(x_ref, o_ref):
    x = x_ref[...]
    m = x[:, 0:RPW]
    s = x[:, RPW:2 * RPW]
    neg = x[:, 2 * RPW:3 * RPW]
    loss = m + jnp.log(s) - neg
    o_ref[0, 0] = jnp.mean(loss)


def _pack2(a, b):
    pa = lax.shift_right_logical(
        lax.bitcast_convert_type(
            a.astype(jnp.bfloat16).astype(jnp.float32), jnp.uint32), jnp.uint32(16))
    pb = jnp.bitwise_and(
        lax.bitcast_convert_type(
            b.astype(jnp.bfloat16).astype(jnp.float32), jnp.uint32),
        jnp.uint32(0xFFFF0000)).astype(jnp.uint32)
    return lax.bitcast_convert_type(pa | pb, jnp.int32)


def _pack_bf16(preds):
    main = _pack2(preds[:, :SPL], preds[:, SPL:2 * SPL])
    tail = _pack2(preds[:, 2 * SPL:2 * SPL + TAIL],
                  preds[:, 2 * SPL + TAIL:])
    return jnp.concatenate([main, tail], axis=1)


def kernel(preds, labels):
    labels32 = labels.astype(jnp.int32)
    stats = _sc_stage(_pack_bf16(preds), labels32)
    out = pl.pallas_call(
        _tc_epilogue_kernel,
        out_shape=jax.ShapeDtypeStruct((1, 1), jnp.float32),
        out_specs=pl.BlockSpec(memory_space=pltpu.SMEM),
    )(stats)
    return out.reshape(())
